# baseline (speedup 1.0000x reference)
import math

import jax
import jax.numpy as jnp
from jax import lax
from jax.experimental import pallas as pl
from jax.experimental.pallas import tpu as pltpu
from jax.experimental.pallas import tpu_sc as plsc

DIM = 64
SCALE = math.sqrt(DIM)

NC = 2
NS = 16
NW = NC * NS

MB = 8
NBUF = 2


def _body(x_hbm, table_hbm, out_hbm, idx_v, rows_v, gsem, ssem):
    bsz, seq = x_hbm.shape
    rows_per_w = bsz // NW
    macros = rows_per_w // MB

    wid = lax.axis_index("s") * NC + lax.axis_index("c")
    brow0 = wid * rows_per_w

    def stage_and_fire(m, b):
        brow = brow0 + m * MB
        pltpu.sync_copy(x_hbm.at[pl.ds(brow, MB)], idx_v.at[b])
        for j in range(MB):
            pltpu.async_copy(
                table_hbm.at[idx_v.at[b, j]],
                rows_v.at[b, j],
                gsem[b],
            )

    def drain_scale_store(m, b):
        brow = brow0 + m * MB
        for j in range(MB):
            pltpu.make_async_copy(
                table_hbm.at[idx_v.at[b, j]],
                rows_v.at[b, j],
                gsem[b],
            ).wait()

        @pl.loop(0, seq, unroll=2)
        def _scale(r):
            for i in range(MB):
                for j in range(DIM // 16):
                    sl = pl.ds(j * 16, 16)
                    rows_v[b, i, r, sl] = rows_v[b, i, r, sl] * SCALE

        pltpu.async_copy(rows_v.at[b], out_hbm.at[pl.ds(brow, MB)], ssem[b])

    def wait_store(m, b):
        brow = brow0 + m * MB
        pltpu.make_async_copy(
            rows_v.at[b], out_hbm.at[pl.ds(brow, MB)], ssem[b]
        ).wait()

    stage_and_fire(0, 0)

    @pl.loop(0, macros, step=NBUF)
    def _macro(m0):
        for b in range(NBUF):
            m = m0 + b
            nxt = m + 1
            nb = (b + 1) % NBUF

            @pl.when(nxt < macros)
            def _fire_next():
                @pl.when(m >= 1)
                def _():
                    wait_store(m - 1, nb)
                stage_and_fire(nxt, nb)

            drain_scale_store(m, b)

    wait_store(macros - 1, (macros - 1) % NBUF)


def kernel(x, table):
    bsz, seq = x.shape

    grid_kernel = pl.kernel(
        _body,
        out_type=jax.ShapeDtypeStruct((bsz, seq, DIM), jnp.float32),
        mesh=plsc.VectorSubcoreMesh(
            core_axis_name="c", subcore_axis_name="s",
            num_cores=NC, num_subcores=NS,
        ),
        scratch_types=[
            pltpu.VMEM((NBUF, MB, seq), jnp.int32),
            pltpu.VMEM((NBUF, MB, seq, DIM), jnp.float32),
            [pltpu.SemaphoreType.DMA] * NBUF,
            [pltpu.SemaphoreType.DMA] * NBUF,
        ],
        compiler_params=pltpu.CompilerParams(use_tc_tiling_on_sc=False),
    )
    return grid_kernel(x, table)
